# baseline (speedup 1.0000x reference)
import functools

import jax
import jax.numpy as jnp
from jax import lax
from jax.experimental import pallas as pl
from jax.experimental.pallas import tpu as pltpu
from jax.experimental.pallas import tpu_sc as plsc

USER_NUM = 100000
EMB = 128

NC = 2
NS = 16
NW = NC * NS

CH = 128
B_PAD = 102400
BPW = B_PAD // NW
NCH = BPW // CH

_mesh = plsc.VectorSubcoreMesh(core_axis_name="c", subcore_axis_name="s")


@functools.partial(
    pl.kernel,
    out_type=jax.ShapeDtypeStruct((B_PAD, EMB), jnp.float32),
    mesh=_mesh,
    scratch_types=[
        pltpu.VMEM((NCH, CH), jnp.int32),
        pltpu.VMEM((CH, EMB), jnp.float32),
        pltpu.SemaphoreType.DMA,
    ],
)
def _gather_kernel(table_hbm, idx_hbm, out_hbm, idx_v, rows_v, gsem):
    wid = lax.axis_index("s") * NC + lax.axis_index("c")
    base = wid * BPW
    pltpu.sync_copy(idx_hbm.at[wid], idx_v)

    @pl.loop(0, NCH)
    def _chunk(i):
        pltpu.async_copy(table_hbm.at[idx_v.at[i]], rows_v, gsem).wait()
        pltpu.sync_copy(rows_v, out_hbm.at[pl.ds(base + i * CH, CH)])


def kernel(user_emb, user_index):
    idx = user_index.astype(jnp.int32)
    idx = jnp.pad(idx, (0, B_PAD - USER_NUM))
    idx3 = idx.reshape(NW, NCH, CH)
    out = _gather_kernel(user_emb, idx3)
    return out[:USER_NUM]


# speedup vs baseline: 1.1525x; 1.1525x over previous
import functools

import jax
import jax.numpy as jnp
from jax import lax
from jax.experimental import pallas as pl
from jax.experimental.pallas import tpu as pltpu
from jax.experimental.pallas import tpu_sc as plsc

USER_NUM = 100000
EMB = 128

NC = 2
NS = 16
NW = NC * NS

CH = 128
B_PAD = 102400
BPW = B_PAD // NW
NCH = BPW // CH
NBUF = 5
LEAD = 2

_mesh = plsc.VectorSubcoreMesh(core_axis_name="c", subcore_axis_name="s")


@functools.partial(
    pl.kernel,
    out_type=jax.ShapeDtypeStruct((B_PAD, EMB), jnp.float32),
    mesh=_mesh,
    scratch_types=[
        pltpu.VMEM((NCH, CH), jnp.int32),
        pltpu.VMEM((NBUF, CH, EMB), jnp.float32),
        [pltpu.SemaphoreType.DMA] * NBUF,
        [pltpu.SemaphoreType.DMA] * NBUF,
    ],
)
def _gather_kernel(table_hbm, idx_hbm, out_hbm, idx_v, rows_v, gs, ws):
    wid = lax.axis_index("s") * NC + lax.axis_index("c")
    base = wid * BPW
    pltpu.sync_copy(idx_hbm.at[wid], idx_v)

    def gather_start(i, b):
        pltpu.async_copy(table_hbm.at[idx_v.at[i]], rows_v.at[b], gs[b])

    def gather_wait(i, b):
        pltpu.make_async_copy(
            table_hbm.at[idx_v.at[i]], rows_v.at[b], gs[b]).wait()

    def wb_start(i, b):
        pltpu.async_copy(
            rows_v.at[b], out_hbm.at[pl.ds(base + i * CH, CH)], ws[b])

    def wb_wait(i, b):
        pltpu.make_async_copy(
            rows_v.at[b], out_hbm.at[pl.ds(base + i * CH, CH)], ws[b]).wait()

    def step(i, b):
        bl = (b + LEAD) % NBUF
        wb_wait(i - (NBUF - LEAD), bl)
        gather_start(i + LEAD, bl)
        gather_wait(i, b)
        wb_start(i, b)

    for k in range(LEAD):
        gather_start(k, k)
    for i in range(NBUF):
        bl = (i + LEAD) % NBUF
        if i - (NBUF - LEAD) >= 0:
            wb_wait(i - (NBUF - LEAD), bl)
        gather_start(i + LEAD, bl)
        gather_wait(i, i)
        wb_start(i, i)

    @pl.loop(NBUF, NCH - NBUF, step=NBUF)
    def _group(i0):
        for k in range(NBUF):
            step(i0 + k, k)

    for i in range(NCH - NBUF, NCH):
        b = i % NBUF
        bl = (b + LEAD) % NBUF
        if i + LEAD < NCH:
            wb_wait(i - (NBUF - LEAD), bl)
            gather_start(i + LEAD, bl)
        gather_wait(i, b)
        wb_start(i, b)

    for i in range(NCH - NBUF, NCH):
        wb_wait(i, i % NBUF)


def kernel(user_emb, user_index):
    idx = user_index.astype(jnp.int32)
    idx = jnp.pad(idx, (0, B_PAD - USER_NUM))
    idx3 = idx.reshape(NW, NCH, CH)
    out = _gather_kernel(user_emb, idx3)
    return out[:USER_NUM]


# speedup vs baseline: 2.5840x; 2.2421x over previous
import functools

import jax
import jax.numpy as jnp
from jax import lax
from jax.experimental import pallas as pl
from jax.experimental.pallas import tpu as pltpu
from jax.experimental.pallas import tpu_sc as plsc

USER_NUM = 100000
EMB = 128

NC = 2
NS = 16
NW = NC * NS

CH = 128
NCH = 25
BPW = NCH * CH
STRIDE = 3128
BASE_MAX = USER_NUM - BPW
NBUF = 5
LEAD = 2

_mesh = plsc.VectorSubcoreMesh(core_axis_name="c", subcore_axis_name="s")


@functools.partial(
    pl.kernel,
    out_type=jax.ShapeDtypeStruct((USER_NUM, EMB), jnp.float32),
    mesh=_mesh,
    scratch_types=[
        pltpu.VMEM((NCH, CH), jnp.int32),
        pltpu.VMEM((NBUF, CH, EMB), jnp.float32),
        [pltpu.SemaphoreType.DMA] * NBUF,
        [pltpu.SemaphoreType.DMA] * NBUF,
    ],
)
def _gather_kernel(table_hbm, idx_hbm, out_hbm, idx_v, rows_v, gs, ws):
    wid = lax.axis_index("s") * NC + (1 - lax.axis_index("c"))
    base = pl.multiple_of(jnp.minimum(wid * STRIDE, BASE_MAX), 8)
    pltpu.sync_copy(idx_hbm.at[wid], idx_v)

    def gather_start(i, b):
        pltpu.async_copy(table_hbm.at[idx_v.at[i]], rows_v.at[b], gs[b])

    def gather_wait(i, b):
        pltpu.make_async_copy(
            table_hbm.at[idx_v.at[i]], rows_v.at[b], gs[b]).wait()

    def wb_start(i, b):
        pltpu.async_copy(
            rows_v.at[b], out_hbm.at[pl.ds(base + i * CH, CH)], ws[b])

    def wb_wait(i, b):
        pltpu.make_async_copy(
            rows_v.at[b], out_hbm.at[pl.ds(base + i * CH, CH)], ws[b]).wait()

    def step(i, b):
        bl = (b + LEAD) % NBUF
        wb_wait(i - (NBUF - LEAD), bl)
        gather_start(i + LEAD, bl)
        gather_wait(i, b)
        wb_start(i, b)

    for k in range(LEAD):
        gather_start(k, k)
    for i in range(NBUF):
        bl = (i + LEAD) % NBUF
        if i - (NBUF - LEAD) >= 0:
            wb_wait(i - (NBUF - LEAD), bl)
        gather_start(i + LEAD, bl)
        gather_wait(i, i)
        wb_start(i, i)

    @pl.loop(NBUF, NCH - NBUF, step=NBUF)
    def _group(i0):
        for k in range(NBUF):
            step(i0 + k, k)

    for i in range(NCH - NBUF, NCH):
        b = i % NBUF
        bl = (b + LEAD) % NBUF
        if i + LEAD < NCH:
            wb_wait(i - (NBUF - LEAD), bl)
            gather_start(i + LEAD, bl)
        gather_wait(i, b)
        wb_start(i, b)

    for i in range(NCH - NBUF, NCH):
        wb_wait(i, i % NBUF)


def kernel(user_emb, user_index):
    idx = user_index.astype(jnp.int32)
    idx3 = jnp.stack([
        lax.slice(idx, (min(w * STRIDE, BASE_MAX),),
                  (min(w * STRIDE, BASE_MAX) + BPW,))
        for w in range(NW)
    ]).reshape(NW, NCH, CH)
    return _gather_kernel(user_emb, idx3)


# speedup vs baseline: 3.7212x; 1.4401x over previous
import functools

import jax
import jax.numpy as jnp
from jax import lax
from jax.experimental import pallas as pl
from jax.experimental.pallas import tpu as pltpu
from jax.experimental.pallas import tpu_sc as plsc

USER_NUM = 100000
EMB = 128

NC = 2
NS = 16
NW = NC * NS

CH = 128
NCH = 25
BPW = NCH * CH
STRIDE = 3128
BASE_MAX = USER_NUM - BPW
NBUF = 6
LEAD = 3

_mesh = plsc.VectorSubcoreMesh(core_axis_name="c", subcore_axis_name="s")


@functools.partial(
    pl.kernel,
    out_type=jax.ShapeDtypeStruct((USER_NUM, EMB), jnp.float32),
    mesh=_mesh,
    scratch_types=[
        pltpu.VMEM((BPW,), jnp.int32),
        pltpu.VMEM((NBUF, CH, EMB), jnp.float32),
        [pltpu.SemaphoreType.DMA] * NBUF,
        [pltpu.SemaphoreType.DMA] * NBUF,
    ],
)
def _gather_kernel(table_hbm, idx_hbm, out_hbm, idx_v, rows_v, gs, ws):
    wid = lax.axis_index("s") * NC + (1 - lax.axis_index("c"))
    base = pl.multiple_of(jnp.minimum(wid * STRIDE, BASE_MAX), 8)
    pltpu.sync_copy(idx_hbm.at[pl.ds(base, BPW)], idx_v)

    def gather_start(i, b):
        pltpu.async_copy(
            table_hbm.at[idx_v.at[pl.ds(i * CH, CH)]], rows_v.at[b], gs[b])

    def gather_wait(i, b):
        pltpu.make_async_copy(
            table_hbm.at[idx_v.at[pl.ds(i * CH, CH)]], rows_v.at[b],
            gs[b]).wait()

    def wb_start(i, b):
        pltpu.async_copy(
            rows_v.at[b], out_hbm.at[pl.ds(base + i * CH, CH)], ws[b])

    def wb_wait(i, b):
        pltpu.make_async_copy(
            rows_v.at[b], out_hbm.at[pl.ds(base + i * CH, CH)], ws[b]).wait()

    for k in range(LEAD):
        gather_start(k, k % NBUF)
    for i in range(NCH):
        b = i % NBUF
        bl = (i + LEAD) % NBUF
        if i + LEAD < NCH:
            if i + LEAD >= NBUF:
                wb_wait(i + LEAD - NBUF, bl)
            gather_start(i + LEAD, bl)
        gather_wait(i, b)
        wb_start(i, b)
    for i in range(max(0, NCH - NBUF), NCH):
        wb_wait(i, i % NBUF)


def kernel(user_emb, user_index):
    return _gather_kernel(user_emb, user_index.astype(jnp.int32))


# speedup vs baseline: 3.7439x; 1.0061x over previous
import functools

import jax
import jax.numpy as jnp
from jax import lax
from jax.experimental import pallas as pl
from jax.experimental.pallas import tpu as pltpu
from jax.experimental.pallas import tpu_sc as plsc

USER_NUM = 100000
EMB = 128

NC = 2
NS = 16
NW = NC * NS

CH = 200
NCH = 16
BPW = NCH * CH
STRIDE = 3128
BASE_MAX = USER_NUM - BPW
NBUF = 4
LEAD = 2

_mesh = plsc.VectorSubcoreMesh(core_axis_name="c", subcore_axis_name="s")


@functools.partial(
    pl.kernel,
    out_type=jax.ShapeDtypeStruct((USER_NUM, EMB), jnp.float32),
    mesh=_mesh,
    scratch_types=[
        pltpu.VMEM((BPW,), jnp.int32),
        pltpu.VMEM((NBUF, CH, EMB), jnp.float32),
        [pltpu.SemaphoreType.DMA] * NBUF,
        [pltpu.SemaphoreType.DMA] * NBUF,
    ],
)
def _gather_kernel(table_hbm, idx_hbm, out_hbm, idx_v, rows_v, gs, ws):
    wid = lax.axis_index("s") * NC + (1 - lax.axis_index("c"))
    base = pl.multiple_of(jnp.minimum(wid * STRIDE, BASE_MAX), 8)
    pltpu.sync_copy(idx_hbm.at[pl.ds(base, BPW)], idx_v)

    def gather_start(i, b):
        pltpu.async_copy(
            table_hbm.at[idx_v.at[pl.ds(i * CH, CH)]], rows_v.at[b], gs[b])

    def gather_wait(i, b):
        pltpu.make_async_copy(
            table_hbm.at[idx_v.at[pl.ds(i * CH, CH)]], rows_v.at[b],
            gs[b]).wait()

    def wb_start(i, b):
        pltpu.async_copy(
            rows_v.at[b], out_hbm.at[pl.ds(base + i * CH, CH)], ws[b])

    def wb_wait(i, b):
        pltpu.make_async_copy(
            rows_v.at[b], out_hbm.at[pl.ds(base + i * CH, CH)], ws[b]).wait()

    for k in range(LEAD):
        gather_start(k, k % NBUF)
    for i in range(NCH):
        b = i % NBUF
        bl = (i + LEAD) % NBUF
        if i + LEAD < NCH:
            if i + LEAD >= NBUF:
                wb_wait(i + LEAD - NBUF, bl)
            gather_start(i + LEAD, bl)
        gather_wait(i, b)
        wb_start(i, b)
    for i in range(max(0, NCH - NBUF), NCH):
        wb_wait(i, i % NBUF)


def kernel(user_emb, user_index):
    return _gather_kernel(user_emb, user_index.astype(jnp.int32))


# speedup vs baseline: 3.7452x; 1.0003x over previous
import functools

import jax
import jax.numpy as jnp
from jax import lax
from jax.experimental import pallas as pl
from jax.experimental.pallas import tpu as pltpu
from jax.experimental.pallas import tpu_sc as plsc

USER_NUM = 100000
EMB = 128

NC = 2
NS = 16
NW = NC * NS

CH = 128
NCH = 25
BPW = NCH * CH
STRIDE = 3128
BASE_MAX = USER_NUM - BPW
NBUF = 7
LEAD = 4

_mesh = plsc.VectorSubcoreMesh(core_axis_name="c", subcore_axis_name="s")


@functools.partial(
    pl.kernel,
    out_type=jax.ShapeDtypeStruct((USER_NUM, EMB), jnp.float32),
    mesh=_mesh,
    scratch_types=[
        pltpu.VMEM((BPW,), jnp.int32),
        pltpu.VMEM((NBUF, CH, EMB), jnp.float32),
        [pltpu.SemaphoreType.DMA] * NBUF,
        [pltpu.SemaphoreType.DMA] * NBUF,
    ],
)
def _gather_kernel(table_hbm, idx_hbm, out_hbm, idx_v, rows_v, gs, ws):
    wid = lax.axis_index("s") * NC + (1 - lax.axis_index("c"))
    base = pl.multiple_of(jnp.minimum(wid * STRIDE, BASE_MAX), 8)
    FIRST = LEAD * CH
    pltpu.sync_copy(idx_hbm.at[pl.ds(base, FIRST)], idx_v.at[pl.ds(0, FIRST)])

    def gather_start(i, b):
        pltpu.async_copy(
            table_hbm.at[idx_v.at[pl.ds(i * CH, CH)]], rows_v.at[b], gs[b])

    def gather_wait(i, b):
        pltpu.make_async_copy(
            table_hbm.at[idx_v.at[pl.ds(i * CH, CH)]], rows_v.at[b],
            gs[b]).wait()

    def wb_start(i, b):
        pltpu.async_copy(
            rows_v.at[b], out_hbm.at[pl.ds(base + i * CH, CH)], ws[b])

    def wb_wait(i, b):
        pltpu.make_async_copy(
            rows_v.at[b], out_hbm.at[pl.ds(base + i * CH, CH)], ws[b]).wait()

    for k in range(LEAD):
        gather_start(k, k % NBUF)
    pltpu.sync_copy(idx_hbm.at[pl.ds(base + LEAD * CH, BPW - LEAD * CH)],
                    idx_v.at[pl.ds(LEAD * CH, BPW - LEAD * CH)])
    for i in range(NCH):
        b = i % NBUF
        bl = (i + LEAD) % NBUF
        if i + LEAD < NCH:
            if i + LEAD >= NBUF:
                wb_wait(i + LEAD - NBUF, bl)
            gather_start(i + LEAD, bl)
        gather_wait(i, b)
        wb_start(i, b)
    for i in range(max(0, NCH - NBUF), NCH):
        wb_wait(i, i % NBUF)


def kernel(user_emb, user_index):
    return _gather_kernel(user_emb, user_index.astype(jnp.int32))
